# baseline (speedup 1.0000x reference)
import functools

import jax
import jax.numpy as jnp
from jax import lax
from jax.experimental import pallas as pl
from jax.experimental.pallas import tpu as pltpu
from jax.experimental.pallas import tpu_sc as plsc

_N = 8192 * 1024
_B = 4
_NC = 2
_NS = 16
_NW = _NC * _NS
_PER_W = _N // _NW
_CH = 32768
_NCH = _PER_W // _CH


def _shuffle_call(flat, perm):
    mesh = plsc.VectorSubcoreMesh(core_axis_name="c", subcore_axis_name="s")

    @functools.partial(
        pl.kernel,
        mesh=mesh,
        out_type=jax.ShapeDtypeStruct((_B, _N), jnp.float32),
        scratch_types=[
            pltpu.VMEM((_CH,), jnp.int32),
            pltpu.VMEM((_CH,), jnp.float32),
            pltpu.SemaphoreType.DMA,
        ],
        compiler_params=pltpu.CompilerParams(use_tc_tiling_on_sc=False),
    )
    def k(flat_hbm, perm_hbm, out_hbm, idx_v, val_v, sem):
        wid = lax.axis_index("s") * _NC + lax.axis_index("c")
        w_base = wid * _PER_W
        for c in range(_NCH):
            base = w_base + c * _CH
            pltpu.sync_copy(perm_hbm.at[pl.ds(base, _CH)], idx_v)
            for b in range(_B):
                pltpu.async_copy(flat_hbm.at[b].at[idx_v], val_v, sem).wait()
                pltpu.sync_copy(val_v, out_hbm.at[b].at[pl.ds(base, _CH)])

    return k(flat, perm)


def kernel(x, perm):
    bsz = x.shape[0]
    dims = x.shape[1:]
    flat = x.reshape(bsz, -1)
    out = _shuffle_call(flat, perm)
    return out.reshape((bsz,) + dims)


# speedup vs baseline: 1.0172x; 1.0172x over previous
import functools

import jax
import jax.numpy as jnp
from jax import lax
from jax.experimental import pallas as pl
from jax.experimental.pallas import tpu as pltpu
from jax.experimental.pallas import tpu_sc as plsc

_N = 8192 * 1024
_B = 4
_NC = 2
_NS = 16
_NW = _NC * _NS
_PER_W = _N // _NW
_CH = 8192
_NCH = _PER_W // _CH


def _shuffle_call(flat, perm):
    mesh = plsc.VectorSubcoreMesh(core_axis_name="c", subcore_axis_name="s")

    @functools.partial(
        pl.kernel,
        mesh=mesh,
        out_type=jax.ShapeDtypeStruct((_B, _N), jnp.float32),
        scratch_types=[
            pltpu.VMEM((2, _CH), jnp.int32),
            pltpu.VMEM((2 * _B, _CH), jnp.float32),
            pltpu.SemaphoreType.DMA,
            pltpu.SemaphoreType.DMA,
        ],
        compiler_params=pltpu.CompilerParams(use_tc_tiling_on_sc=False),
    )
    def k(flat_hbm, perm_hbm, out_hbm, idx_v, val_v, sem_g, sem_s):
        wid = lax.axis_index("s") * _NC + lax.axis_index("c")
        w_base = wid * _PER_W

        def base(c):
            return w_base + c * _CH

        def idx_ref(c):
            return idx_v.at[c % 2]

        def val_ref(c, b):
            return val_v.at[(c % 2) * _B + b]

        def start_gathers(c):
            return [
                pltpu.async_copy(flat_hbm.at[b].at[idx_ref(c)],
                                 val_ref(c, b), sem_g)
                for b in range(_B)
            ]

        def start_scatters(c):
            return [
                pltpu.async_copy(val_ref(c, b),
                                 out_hbm.at[b].at[pl.ds(base(c), _CH)],
                                 sem_s)
                for b in range(_B)
            ]

        gathers = {}
        scatters = {}
        pltpu.sync_copy(perm_hbm.at[pl.ds(base(0), _CH)], idx_ref(0))
        gathers[0] = start_gathers(0)
        pltpu.sync_copy(perm_hbm.at[pl.ds(base(1), _CH)], idx_ref(1))
        for c in range(_NCH):
            for cp in gathers.pop(c):
                cp.wait()
            if c >= 1:
                for cp in scatters.pop(c - 1):
                    cp.wait()
            scatters[c] = start_scatters(c)
            if c + 1 < _NCH:
                gathers[c + 1] = start_gathers(c + 1)
            if c + 2 < _NCH:
                pltpu.sync_copy(perm_hbm.at[pl.ds(base(c + 2), _CH)],
                                idx_ref(c + 2))
        for cp in scatters.pop(_NCH - 1):
            cp.wait()

    return k(flat, perm)


def kernel(x, perm):
    bsz = x.shape[0]
    dims = x.shape[1:]
    flat = x.reshape(bsz, -1)
    out = _shuffle_call(flat, perm)
    return out.reshape((bsz,) + dims)
